# baseline (speedup 1.0000x reference)
import functools

import jax
import jax.numpy as jnp
from jax import lax
from jax.experimental import pallas as pl
from jax.experimental.pallas import tpu as pltpu
from jax.experimental.pallas import tpu_sc as plsc

_ITERS = 10
_TOPK = 4


def _kmeans_topk_kernel(p1_ref, c0_ref, trans_ref, protos_ref, idx_ref):
    n = p1_ref.shape[0]
    k = c0_ref.shape[1]

    p1 = p1_ref[:, :]
    pts = p1[:, 0:3]
    a2 = jnp.sum(pts * pts, axis=1, keepdims=True)

    def dist(cT):
        b2 = jnp.sum(cT * cT, axis=0, keepdims=True)
        ab = lax.dot_general(pts, cT, (((1,), (0,)), ((), ())),
                             preferred_element_type=jnp.float32)
        return jnp.sqrt(jnp.maximum(a2 + b2 - 2.0 * ab, 0.0))

    def step(_, cT):
        d = dist(cT)
        rmin = jnp.min(d, axis=1, keepdims=True)
        il = lax.broadcasted_iota(jnp.int32, (n, k), 1)
        amin = jnp.min(jnp.where(d == rmin, il, k), axis=1, keepdims=True)
        oh = (il == amin).astype(jnp.float32)
        cnt = jnp.sum(oh, axis=0, keepdims=True)
        sx = jnp.sum(oh * p1[:, 0:1], axis=0, keepdims=True)
        sy = jnp.sum(oh * p1[:, 1:2], axis=0, keepdims=True)
        sz = jnp.sum(oh * p1[:, 2:3], axis=0, keepdims=True)
        sums = jnp.concatenate([sx, sy, sz], axis=0)
        return jnp.where(cnt > 0, sums / jnp.maximum(cnt, 1.0), cT)

    cT = lax.fori_loop(0, _ITERS, step, c0_ref[:, :])

    d = dist(cT)

    cmin = jnp.min(d, axis=0, keepdims=True)
    isrc = lax.broadcasted_iota(jnp.int32, (n, k), 0)
    nearest = jnp.min(jnp.where(d == cmin, isrc, n), axis=0, keepdims=True)
    oh_n = (isrc == nearest).astype(jnp.bfloat16)
    protos_ref[:, :] = lax.dot_general(
        oh_n, trans_ref[:, :], (((0,), (0,)), ((), ())),
        preferred_element_type=jnp.float32)

    dt = d[0:k, :]
    il9 = lax.broadcasted_iota(jnp.int32, (k, k), 1)
    for t in range(_TOPK):
        rmin = jnp.min(dt, axis=1, keepdims=True)
        amin = jnp.min(jnp.where(dt == rmin, il9, k), axis=1, keepdims=True)
        idx_ref[:, t:t + 1] = amin
        if t + 1 < _TOPK:
            dt = jnp.where(il9 == amin, jnp.float32(jnp.inf), dt)


def _sc_gather_sum(table, idx_pad, n_out_pad):
    info = plsc.get_sparse_core_info()
    nw = info.num_cores * info.num_subcores
    lanes = info.num_lanes
    depth = table.shape[1]
    bpw = idx_pad.shape[0] // nw
    opw = bpw // _TOPK
    mesh = plsc.VectorSubcoreMesh(core_axis_name="c", subcore_axis_name="s")

    @functools.partial(
        pl.kernel, mesh=mesh,
        out_type=jax.ShapeDtypeStruct((n_out_pad, depth), jnp.float32),
        scratch_types=[
            pltpu.VMEM((bpw,), jnp.int32),
            pltpu.VMEM((bpw, depth), jnp.float32),
            pltpu.VMEM((opw, depth), jnp.float32),
            pltpu.SemaphoreType.DMA,
        ],
    )
    def k(table_hbm, idx_hbm, out_hbm, idx_v, rows_v, acc_v, sem):
        wid = lax.axis_index("s") * info.num_cores + lax.axis_index("c")
        base = wid * bpw
        pltpu.sync_copy(idx_hbm.at[pl.ds(base, bpw)], idx_v)
        pltpu.async_copy(table_hbm.at[idx_v], rows_v, sem).wait()

        def body(o, carry):
            r = o * _TOPK
            for c in range(depth // lanes):
                sl = pl.ds(c * lanes, lanes)
                s = ((rows_v[r, sl] + rows_v[r + 1, sl])
                     + rows_v[r + 2, sl]) + rows_v[r + 3, sl]
                acc_v[o, sl] = s
            return carry

        lax.fori_loop(0, opw, body, 0)
        pltpu.sync_copy(acc_v, out_hbm.at[pl.ds(wid * opw, opw)])

    return k(table, idx_pad)


def kernel(ego_anchor, trans_anchor, ego_feature, instance_feature):
    N, A, D = trans_anchor.shape
    E = instance_feature.shape[-1]
    trans_flat = trans_anchor.reshape(N * A, D)
    pts = trans_flat[:, :3]
    p1 = jnp.concatenate([pts, jnp.ones((N * A, 1), jnp.float32)], axis=1)
    c0T = jnp.transpose(pts[:: (N * A) // A])
    inst0 = instance_feature.reshape(N * A, E)[:A]

    protos, idx = pl.pallas_call(
        _kmeans_topk_kernel,
        out_shape=(jax.ShapeDtypeStruct((A, D), jnp.float32),
                   jax.ShapeDtypeStruct((A, _TOPK), jnp.int32)),
    )(p1, c0T, trans_flat)

    n_out_pad = 1024
    idx_pad = jnp.zeros((n_out_pad * _TOPK,), jnp.int32)
    idx_pad = lax.dynamic_update_slice(idx_pad, idx.reshape(A * _TOPK), (0,))
    fused = _sc_gather_sum(inst0, idx_pad, n_out_pad)[:A]
    return protos, fused


# speedup vs baseline: 1.2030x; 1.2030x over previous
import jax
import jax.numpy as jnp
from jax import lax
from jax.experimental import pallas as pl

_ITERS = 10
_TOPK = 4


def _kmeans_fusion_kernel(pts_ref, c0_ref, trans_ref, inst_ref,
                          protos_ref, fused_ref):
    n = pts_ref.shape[0]
    k = c0_ref.shape[1]

    pts = pts_ref[:, :]
    px = pts[:, 0:1]
    py = pts[:, 1:2]
    pz = pts[:, 2:3]
    a2 = jnp.sum(pts * pts, axis=1, keepdims=True)

    def dist(cT):
        b2 = jnp.sum(cT * cT, axis=0, keepdims=True)
        ab = lax.dot_general(pts, cT, (((1,), (0,)), ((), ())),
                             preferred_element_type=jnp.float32)
        return jnp.sqrt(jnp.maximum(a2 + b2 - 2.0 * ab, 0.0))

    def step(_, cT):
        d = dist(cT)
        rmin = jnp.min(d, axis=1, keepdims=True)
        il = lax.broadcasted_iota(jnp.int32, (n, k), 1)
        amin = jnp.min(jnp.where(d == rmin, il, k), axis=1, keepdims=True)
        oh = (il == amin).astype(jnp.float32)
        cnt = jnp.sum(oh, axis=0, keepdims=True)
        sx = jnp.sum(oh * px, axis=0, keepdims=True)
        sy = jnp.sum(oh * py, axis=0, keepdims=True)
        sz = jnp.sum(oh * pz, axis=0, keepdims=True)
        sums = jnp.concatenate([sx, sy, sz], axis=0)
        return jnp.where(cnt > 0, sums / jnp.maximum(cnt, 1.0), cT)

    cT = lax.fori_loop(0, _ITERS, step, c0_ref[:, :])

    d = dist(cT)

    cmin = jnp.min(d, axis=0, keepdims=True)
    isrc = lax.broadcasted_iota(jnp.int32, (n, k), 0)
    nearest = jnp.min(jnp.where(d == cmin, isrc, n), axis=0, keepdims=True)
    oh_n = (isrc == nearest).astype(jnp.bfloat16)
    protos_ref[:, :] = lax.dot_general(
        oh_n, trans_ref[:, :], (((0,), (0,)), ((), ())),
        preferred_element_type=jnp.float32)

    dt = d[0:k, :]
    il9 = lax.broadcasted_iota(jnp.int32, (k, k), 1)

    def tstep(_, carry):
        w, dcur = carry
        rmin = jnp.min(dcur, axis=1, keepdims=True)
        amin = jnp.min(jnp.where(dcur == rmin, il9, k), axis=1, keepdims=True)
        sel = (il9 == amin)
        return (w + sel.astype(jnp.float32),
                jnp.where(sel, jnp.float32(jnp.inf), dcur))

    w0 = jnp.zeros((k, k), jnp.float32)
    w, _ = lax.fori_loop(0, _TOPK, tstep, (w0, dt))
    fused_ref[:, :] = lax.dot_general(
        w.astype(jnp.bfloat16), inst_ref[:, :], (((1,), (0,)), ((), ())),
        preferred_element_type=jnp.float32)


def kernel(ego_anchor, trans_anchor, ego_feature, instance_feature):
    N, A, D = trans_anchor.shape
    E = instance_feature.shape[-1]
    trans_flat = trans_anchor.reshape(N * A, D)
    pts = trans_flat[:, :3]
    c0T = jnp.transpose(pts[:: (N * A) // A])
    inst0 = instance_feature.reshape(N * A, E)[:A]

    protos, fused = pl.pallas_call(
        _kmeans_fusion_kernel,
        out_shape=(jax.ShapeDtypeStruct((A, D), jnp.float32),
                   jax.ShapeDtypeStruct((A, E), jnp.float32)),
    )(pts, c0T, trans_flat, inst0)
    return protos, fused


# speedup vs baseline: 1.2873x; 1.0701x over previous
import jax
import jax.numpy as jnp
from jax import lax
from jax.experimental import pallas as pl

_ITERS = 10
_TOPK = 4


def _kmeans_fusion_kernel(pts_ref, c0_ref, trans_ref, inst_ref,
                          protos_ref, fused_ref):
    n = pts_ref.shape[0]
    k = c0_ref.shape[1]

    pts = pts_ref[:, :]
    px = pts[:, 0:1]
    py = pts[:, 1:2]
    pz = pts[:, 2:3]
    a2 = jnp.sum(pts * pts, axis=1, keepdims=True)

    def dist(cT):
        b2 = jnp.sum(cT * cT, axis=0, keepdims=True)
        ab = lax.dot_general(pts, cT, (((1,), (0,)), ((), ())),
                             preferred_element_type=jnp.float32)
        return jnp.sqrt(jnp.maximum(a2 + b2 - 2.0 * ab, 0.0))

    def step(_, cT):
        d = dist(cT)
        il = lax.broadcasted_iota(jnp.int32, (n, k), 1)
        amin = jnp.argmin(d, axis=1).astype(jnp.int32)[:, None]
        oh = (il == amin).astype(jnp.float32)
        cnt = jnp.sum(oh, axis=0, keepdims=True)
        sx = jnp.sum(oh * px, axis=0, keepdims=True)
        sy = jnp.sum(oh * py, axis=0, keepdims=True)
        sz = jnp.sum(oh * pz, axis=0, keepdims=True)
        sums = jnp.concatenate([sx, sy, sz], axis=0)
        return jnp.where(cnt > 0, sums / jnp.maximum(cnt, 1.0), cT)

    cT = lax.fori_loop(0, _ITERS, step, c0_ref[:, :])

    d = dist(cT)

    cmin = jnp.min(d, axis=0, keepdims=True)
    isrc = lax.broadcasted_iota(jnp.int32, (n, k), 0)
    nearest = jnp.min(jnp.where(d == cmin, isrc, n), axis=0, keepdims=True)
    oh_n = (isrc == nearest).astype(jnp.bfloat16)
    protos_ref[:, :] = lax.dot_general(
        oh_n, trans_ref[:, :], (((0,), (0,)), ((), ())),
        preferred_element_type=jnp.float32)

    dt = d[0:k, :]
    il9 = lax.broadcasted_iota(jnp.int32, (k, k), 1)

    def tstep(_, carry):
        w, dcur = carry
        rmin = jnp.min(dcur, axis=1, keepdims=True)
        amin = jnp.min(jnp.where(dcur == rmin, il9, k), axis=1, keepdims=True)
        sel = (il9 == amin)
        return (w + sel.astype(jnp.float32),
                jnp.where(sel, jnp.float32(jnp.inf), dcur))

    w0 = jnp.zeros((k, k), jnp.float32)
    w, _ = lax.fori_loop(0, _TOPK, tstep, (w0, dt))
    fused_ref[:, :] = lax.dot_general(
        w.astype(jnp.bfloat16), inst_ref[:, :], (((1,), (0,)), ((), ())),
        preferred_element_type=jnp.float32)


def kernel(ego_anchor, trans_anchor, ego_feature, instance_feature):
    N, A, D = trans_anchor.shape
    E = instance_feature.shape[-1]
    trans_flat = trans_anchor.reshape(N * A, D)
    pts = trans_flat[:, :3]
    c0T = jnp.transpose(pts[:: (N * A) // A])
    inst0 = instance_feature.reshape(N * A, E)[:A]

    protos, fused = pl.pallas_call(
        _kmeans_fusion_kernel,
        out_shape=(jax.ShapeDtypeStruct((A, D), jnp.float32),
                   jax.ShapeDtypeStruct((A, E), jnp.float32)),
    )(pts, c0T, trans_flat, inst0)
    return protos, fused


# speedup vs baseline: 1.3464x; 1.0459x over previous
import jax
import jax.numpy as jnp
from jax import lax
from jax.experimental import pallas as pl

_ITERS = 10
_TOPK = 4


def _kmeans_fusion_kernel(pts_ref, hi_ref, mid_ref, lo_ref, c0_ref,
                          trans_ref, inst_ref, protos_ref, fused_ref):
    n = pts_ref.shape[0]
    k = c0_ref.shape[1]

    pts = pts_ref[:, :]
    a2 = jnp.sum(pts * pts, axis=1, keepdims=True)

    def dist(cT):
        b2 = jnp.sum(cT * cT, axis=0, keepdims=True)
        ab = lax.dot_general(pts, cT, (((1,), (0,)), ((), ())),
                             preferred_element_type=jnp.float32)
        return jnp.sqrt(jnp.maximum(a2 + b2 - 2.0 * ab, 0.0))

    def seg_dot(part, oh):
        return lax.dot_general(part, oh, (((0,), (0,)), ((), ())),
                               preferred_element_type=jnp.float32)

    def step(_, cT):
        d = dist(cT)
        rmin = jnp.min(d, axis=1, keepdims=True)
        il = lax.broadcasted_iota(jnp.int32, (n, k), 1)
        amin = jnp.min(jnp.where(d == rmin, il, k), axis=1, keepdims=True)
        oh = (il == amin).astype(jnp.bfloat16)
        cc = (seg_dot(hi_ref[:, :], oh)
              + seg_dot(mid_ref[:, :], oh)) + seg_dot(lo_ref[:, :], oh)
        sums = cc[0:3, :]
        cnt = cc[3:4, :]
        return jnp.where(cnt > 0, sums / jnp.maximum(cnt, 1.0), cT)

    cT = lax.fori_loop(0, _ITERS, step, c0_ref[:, :])

    d = dist(cT)

    cmin = jnp.min(d, axis=0, keepdims=True)
    isrc = lax.broadcasted_iota(jnp.int32, (n, k), 0)
    nearest = jnp.min(jnp.where(d == cmin, isrc, n), axis=0, keepdims=True)
    oh_n = (isrc == nearest).astype(jnp.bfloat16)
    protos_ref[:, :] = lax.dot_general(
        oh_n, trans_ref[:, :], (((0,), (0,)), ((), ())),
        preferred_element_type=jnp.float32)

    dt = d[0:k, :]
    il9 = lax.broadcasted_iota(jnp.int32, (k, k), 1)

    def tstep(_, carry):
        w, dcur = carry
        rmin = jnp.min(dcur, axis=1, keepdims=True)
        amin = jnp.min(jnp.where(dcur == rmin, il9, k), axis=1, keepdims=True)
        sel = (il9 == amin)
        return (w + sel.astype(jnp.float32),
                jnp.where(sel, jnp.float32(jnp.inf), dcur))

    w0 = jnp.zeros((k, k), jnp.float32)
    w, _ = lax.fori_loop(0, _TOPK, tstep, (w0, dt))
    fused_ref[:, :] = lax.dot_general(
        w.astype(jnp.bfloat16), inst_ref[:, :], (((1,), (0,)), ((), ())),
        preferred_element_type=jnp.float32)


def kernel(ego_anchor, trans_anchor, ego_feature, instance_feature):
    N, A, D = trans_anchor.shape
    E = instance_feature.shape[-1]
    trans_flat = trans_anchor.reshape(N * A, D)
    pts = trans_flat[:, :3]
    c0T = jnp.transpose(pts[:: (N * A) // A])
    inst0 = instance_feature.reshape(N * A, E)[:A]

    p1 = jnp.concatenate([pts, jnp.ones((N * A, 1), jnp.float32)], axis=1)
    hi = p1.astype(jnp.bfloat16)
    r1 = p1 - hi.astype(jnp.float32)
    mid = r1.astype(jnp.bfloat16)
    lo = (r1 - mid.astype(jnp.float32)).astype(jnp.bfloat16)

    protos, fused = pl.pallas_call(
        _kmeans_fusion_kernel,
        out_shape=(jax.ShapeDtypeStruct((A, D), jnp.float32),
                   jax.ShapeDtypeStruct((A, E), jnp.float32)),
    )(pts, hi, mid, lo, c0T, trans_flat, inst0)
    return protos, fused
